# baseline (speedup 1.0000x reference)
import functools

import jax
import jax.numpy as jnp
from jax import lax
from jax.experimental import pallas as pl
from jax.experimental.pallas import tpu as pltpu
from jax.experimental.pallas import tpu_sc as plsc

_L = 16
_CHUNK = 128


def _make_sc_call(batch, n_factors):
    info = plsc.get_sparse_core_info()
    nc, ns = info.num_cores, info.num_subcores
    nw = nc * ns
    b_per_w = batch // nw
    n_chunks = b_per_w // _CHUNK
    n_groups = b_per_w // _L
    mesh = plsc.VectorSubcoreMesh(core_axis_name="c", subcore_axis_name="s")

    @functools.partial(
        pl.kernel,
        out_type=jax.ShapeDtypeStruct((nw, b_per_w), jnp.float32),
        mesh=mesh,
        compiler_params=pltpu.CompilerParams(
            needs_layout_passes=False, use_tc_tiling_on_sc=False),
        scratch_types=dict(
            uidx_v=pltpu.VMEM((n_chunks, _CHUNK), jnp.int32),
            aidx_v=pltpu.VMEM((n_chunks, _CHUNK), jnp.int32),
            urows_v=pltpu.VMEM((b_per_w, n_factors), jnp.float32),
            arows_v=pltpu.VMEM((b_per_w, n_factors), jnp.float32),
            ubias_v=pltpu.VMEM((b_per_w,), jnp.float32),
            abias_v=pltpu.VMEM((b_per_w,), jnp.float32),
            out_v=pltpu.VMEM((b_per_w,), jnp.float32),
            sem=pltpu.SemaphoreType.DMA,
        ),
    )
    def sc_call(uidx_hbm, aidx_hbm, uf_hbm, af_hbm, ub_hbm, ab_hbm, out_hbm,
                uidx_v, aidx_v, urows_v, arows_v, ubias_v, abias_v, out_v,
                sem):
        wid = lax.axis_index("s") * nc + lax.axis_index("c")

        pltpu.sync_copy(uidx_hbm.at[wid], uidx_v)
        pltpu.sync_copy(aidx_hbm.at[wid], aidx_v)

        copies = []
        for i in range(n_chunks):
            copies.append(pltpu.async_copy(
                uf_hbm.at[uidx_v.at[i]],
                urows_v.at[pl.ds(i * _CHUNK, _CHUNK)], sem))
            copies.append(pltpu.async_copy(
                af_hbm.at[aidx_v.at[i]],
                arows_v.at[pl.ds(i * _CHUNK, _CHUNK)], sem))
            copies.append(pltpu.async_copy(
                ub_hbm.at[uidx_v.at[i]],
                ubias_v.at[pl.ds(i * _CHUNK, _CHUNK)], sem))
            copies.append(pltpu.async_copy(
                ab_hbm.at[aidx_v.at[i]],
                abias_v.at[pl.ds(i * _CHUNK, _CHUNK)], sem))
        for c in copies:
            c.wait()

        lane = lax.iota(jnp.int32, _L)
        n_col_chunks = n_factors // _L

        def gbody(g, _):
            base = g * _L
            res = jnp.zeros((_L,), jnp.float32)
            for rr in range(_L):
                row = base + rr
                acc = (urows_v[row, pl.ds(0, _L)]
                       * arows_v[row, pl.ds(0, _L)])
                for c in range(1, n_col_chunks):
                    acc += (urows_v[row, pl.ds(c * _L, _L)]
                            * arows_v[row, pl.ds(c * _L, _L)])
                res = jnp.where(lane == rr, jnp.sum(acc), res)
            r = res + ubias_v[pl.ds(base, _L)] + abias_v[pl.ds(base, _L)]
            out_v[pl.ds(base, _L)] = 10.5 / (1.0 + jnp.exp(-r))
            return 0

        lax.fori_loop(0, n_groups, gbody, 0)

        pltpu.sync_copy(out_v, out_hbm.at[wid])

    return sc_call


def kernel(x, user_factors, anime_factors, user_bias, anime_bias):
    batch = x.shape[0]
    n_factors = user_factors.shape[1]
    info = plsc.get_sparse_core_info()
    nw = info.num_cores * info.num_subcores
    b_per_w = batch // nw
    n_chunks = b_per_w // _CHUNK

    uidx = x[:, 0].reshape(nw, n_chunks, _CHUNK)
    aidx = x[:, 1].reshape(nw, n_chunks, _CHUNK)
    ub = user_bias.reshape(-1)
    ab = anime_bias.reshape(-1)

    sc_call = _make_sc_call(batch, n_factors)
    out = sc_call(uidx, aidx, user_factors, anime_factors, ub, ab)
    return out.reshape(batch, 1)
